# Initial kernel scaffold; baseline (speedup 1.0000x reference)
import jax
import jax.numpy as jnp
from jax.experimental import pallas as pl


def kernel(x, edge_index, batch, W1l, W1r, b1, g1, be1, W2l, W2r, b2, g2, be2, W3l, W3r, b3, g3, be3, fcW1, fcb1, fcW2, fcb2):
    raise NotImplementedError("write your pallas kernel here")


# speedup vs baseline: 8.7001x; 8.7001x over previous
import functools

import jax
import jax.numpy as jnp
from jax import lax
from jax.experimental import pallas as pl
from jax.experimental.pallas import tpu as pltpu
from jax.experimental.pallas import tpu_sc as plsc

N = 10000
E = 320000
G = 32
C = 10
EPS = 1e-5

NC = 2
NS = 16
NW = NC * NS
EPW = E // NW
CH = 80
NCH = EPW // CH
NPAD = 10240
RPS = NPAD // NS
RB = 1000
GRID = N // RB


def _sc_agg(t, src, dst, D, with_cnt):
    mesh = plsc.VectorSubcoreMesh(core_axis_name="c", subcore_axis_name="s")
    out_type = [jax.ShapeDtypeStruct((NC, NPAD, D), jnp.float32)]
    scratch = [
        pltpu.VMEM((CH,), jnp.int32),
        pltpu.VMEM((CH,), jnp.int32),
        pltpu.VMEM((CH,), jnp.int32),
        pltpu.VMEM((CH,), jnp.int32),
        pltpu.VMEM((CH, D), jnp.float32),
        pltpu.VMEM((CH, D), jnp.float32),
        pltpu.VMEM_SHARED((NPAD, D), jnp.float32),
        pltpu.SemaphoreType.DMA,
        pltpu.SemaphoreType.DMA,
    ]
    if with_cnt:
        out_type.append(jax.ShapeDtypeStruct((NC, NS, NPAD), jnp.float32))
        scratch.append(pltpu.VMEM((NPAD,), jnp.float32))

    def body(t_hbm, src_hbm, dst_hbm, z_hbm, *rest):
        if with_cnt:
            (zc_hbm, out_hbm, cnt_hbm, isa, ida, isb, idb, rowsa, rowsb,
             acc, sema, semb, cloc) = rest
        else:
            (out_hbm, isa, ida, isb, idb, rowsa, rowsb,
             acc, sema, semb) = rest
        c = lax.axis_index("c")
        s = lax.axis_index("s")
        wid = s * NC + c
        r0 = s * RPS
        base = wid * EPW

        def load_idx(i_s, i_d, off):
            pltpu.sync_copy(src_hbm.at[pl.ds(off, CH)], i_s)
            pltpu.sync_copy(dst_hbm.at[pl.ds(off, CH)], i_d)

        def fire(i_s, rows, sem):
            pltpu.async_copy(t_hbm.at[i_s], rows, sem)

        def drain(rows, sem):
            pltpu.make_async_copy(t_hbm.at[pl.ds(0, CH)], rows, sem).wait()

        def commit(rows, i_d):
            pltpu.sync_copy(rows, acc.at[i_d], add=True)
            if with_cnt:
                for j in range(CH // 16):
                    iv = i_d[pl.ds(j * 16, 16)]
                    cnts, lastm = plsc.scan_count(iv)
                    plsc.addupdate_scatter(cloc, [iv],
                                           cnts.astype(jnp.float32),
                                           mask=lastm)

        for k in range(RPS // CH):
            pltpu.sync_copy(z_hbm.at[pl.ds(r0 + k * CH, CH)], rowsa)
            pltpu.sync_copy(rowsa, acc.at[pl.ds(r0 + k * CH, CH)])
        if with_cnt:
            pltpu.sync_copy(zc_hbm, cloc)
        plsc.subcore_barrier()

        load_idx(isa, ida, base)
        fire(isa, rowsa, sema)

        @pl.loop(0, (NCH - 1) // 2)
        def _pair(k):
            off_b = pl.multiple_of(base + (2 * k + 1) * CH, 8)
            off_a2 = pl.multiple_of(base + (2 * k + 2) * CH, 8)
            load_idx(isb, idb, off_b)
            drain(rowsa, sema)
            fire(isb, rowsb, semb)
            commit(rowsa, ida)
            load_idx(isa, ida, off_a2)
            drain(rowsb, semb)
            fire(isa, rowsa, sema)
            commit(rowsb, idb)

        drain(rowsa, sema)
        commit(rowsa, ida)

        plsc.subcore_barrier()

        for k in range(RPS // CH):
            pltpu.sync_copy(acc.at[pl.ds(r0 + k * CH, CH)], rowsa)
            pltpu.sync_copy(rowsa, out_hbm.at[c, pl.ds(r0 + k * CH, CH)])
        if with_cnt:
            pltpu.sync_copy(cloc, cnt_hbm.at[c, s])

    f = pl.kernel(body, out_type=tuple(out_type), mesh=mesh,
                  scratch_types=tuple(scratch),
                  compiler_params=pltpu.CompilerParams(
                      needs_layout_passes=False))
    zargs = (jnp.zeros((NPAD, D), jnp.float32),)
    if with_cnt:
        zargs += (jnp.zeros((NPAD,), jnp.float32),)
    return f(t, src, dst, *zargs)


def _row_spec(width):
    return pl.BlockSpec((RB, width), lambda i: (i, 0))


def _full_spec(shape):
    return pl.BlockSpec(shape, lambda i: tuple(0 for _ in shape))


def _tc_a1_body(aggp, cntp, x, W1l, W1r, b1, p_out, cnt_out, s_out, q_out):
    i = pl.program_id(0)
    cnt = jnp.sum(cntp[...], axis=1, keepdims=True)
    cnt_out[...] = cnt
    mean = (aggp[0] + aggp[1]) / jnp.maximum(cnt, 1.0)
    p = (jnp.dot(mean, W1l[...], preferred_element_type=jnp.float32)
         + jnp.dot(x[...], W1r[...], preferred_element_type=jnp.float32)
         + b1[...])
    p_out[...] = p
    ps = jnp.sum(p, axis=0, keepdims=True)
    qs = jnp.sum(p * p, axis=0, keepdims=True)

    @pl.when(i == 0)
    def _():
        s_out[...] = jnp.zeros_like(s_out)
        q_out[...] = jnp.zeros_like(q_out)

    s_out[...] += ps
    q_out[...] += qs


def _tc_a23_body(aggp, cnt, u, p_out, s_out, q_out, *, D):
    i = pl.program_id(0)
    mean = (aggp[0][:, :D] + aggp[1][:, :D]) / jnp.maximum(cnt[...], 1.0)
    p = mean + u[...]
    p_out[...] = p

    @pl.when(i == 0)
    def _():
        s_out[...] = jnp.zeros_like(s_out)
        q_out[...] = jnp.zeros_like(q_out)

    s_out[...] += jnp.sum(p, axis=0, keepdims=True)
    q_out[...] += jnp.sum(p * p, axis=0, keepdims=True)


def _tc_b_body(p_in, s_in, q_in, g, be, Wl, Wr, b, t_out, u_out):
    mu = s_in[...] * (1.0 / N)
    var = q_in[...] * (1.0 / N) - mu * mu
    h = (p_in[...] - mu) * lax.rsqrt(var + EPS) * g[...] + be[...]
    h = jnp.maximum(h, 0.0)
    t_out[...] = jnp.dot(h, Wl[...], preferred_element_type=jnp.float32)
    u_out[...] = jnp.dot(h, Wr[...], preferred_element_type=jnp.float32) + b[...]


def _tc_c_body(p_in, s_in, q_in, g, be, batch, fcW1, fcb1, fcW2, fcb2,
               out, ps_acc, pc_acc):
    i = pl.program_id(0)
    mu = s_in[...] * (1.0 / N)
    var = q_in[...] * (1.0 / N) - mu * mu
    h = (p_in[...] - mu) * lax.rsqrt(var + EPS) * g[...] + be[...]
    h = jnp.maximum(h, 0.0)
    gids = lax.broadcasted_iota(jnp.int32, (G, RB), 0)
    oht = (gids == batch[0]).astype(jnp.float32)

    @pl.when(i == 0)
    def _():
        ps_acc[...] = jnp.zeros_like(ps_acc)
        pc_acc[...] = jnp.zeros_like(pc_acc)

    ps_acc[...] += jnp.dot(oht, h, preferred_element_type=jnp.float32)
    pc_acc[...] += jnp.sum(oht, axis=1, keepdims=True)

    @pl.when(i == GRID - 1)
    def _():
        pooled = ps_acc[...] / jnp.maximum(pc_acc[...], 1.0)
        hf = jnp.maximum(
            jnp.dot(pooled, fcW1[...], preferred_element_type=jnp.float32)
            + fcb1[...], 0.0)
        out[...] = (jnp.dot(hf, fcW2[...], preferred_element_type=jnp.float32)
                    + fcb2[...])


def _tc_a1(aggp, cntp, x, W1l, W1r, b1):
    return pl.pallas_call(
        _tc_a1_body,
        grid=(GRID,),
        in_specs=[
            pl.BlockSpec((NC, RB, 128), lambda i: (0, i, 0)),
            pl.BlockSpec((RB, NW), lambda i: (i, 0)),
            _row_spec(128),
            _full_spec((128, 256)), _full_spec((128, 256)), _full_spec((1, 256)),
        ],
        out_specs=[
            _row_spec(256), _row_spec(1),
            _full_spec((1, 256)), _full_spec((1, 256)),
        ],
        out_shape=[
            jax.ShapeDtypeStruct((N, 256), jnp.float32),
            jax.ShapeDtypeStruct((N, 1), jnp.float32),
            jax.ShapeDtypeStruct((1, 256), jnp.float32),
            jax.ShapeDtypeStruct((1, 256), jnp.float32),
        ],
    )(aggp, cntp, x, W1l, W1r, b1)


def _tc_a23(aggp, cnt, u, Dagg, D):
    return pl.pallas_call(
        functools.partial(_tc_a23_body, D=D),
        grid=(GRID,),
        in_specs=[
            pl.BlockSpec((NC, RB, Dagg), lambda i: (0, i, 0)),
            _row_spec(1), _row_spec(D),
        ],
        out_specs=[_row_spec(D), _full_spec((1, D)), _full_spec((1, D))],
        out_shape=[
            jax.ShapeDtypeStruct((N, D), jnp.float32),
            jax.ShapeDtypeStruct((1, D), jnp.float32),
            jax.ShapeDtypeStruct((1, D), jnp.float32),
        ],
    )(aggp, cnt, u)


def _tc_b(p, s, q, g, be, Wl, Wr, b, Din, Dt, Du):
    return pl.pallas_call(
        _tc_b_body,
        grid=(GRID,),
        in_specs=[
            _row_spec(Din),
            _full_spec((1, Din)), _full_spec((1, Din)),
            _full_spec((1, Din)), _full_spec((1, Din)),
            _full_spec((Din, Dt)), _full_spec((Din, Du)),
            _full_spec((1, Du)),
        ],
        out_specs=[_row_spec(Dt), _row_spec(Du)],
        out_shape=[
            jax.ShapeDtypeStruct((N, Dt), jnp.float32),
            jax.ShapeDtypeStruct((N, Du), jnp.float32),
        ],
    )(p, s, q, g, be, Wl, Wr, b)


def _tc_c(p, s, q, g, be, batch2d, fcW1, fcb1, fcW2, fcb2):
    return pl.pallas_call(
        _tc_c_body,
        grid=(GRID,),
        in_specs=[
            _row_spec(64),
            _full_spec((1, 64)), _full_spec((1, 64)),
            _full_spec((1, 64)), _full_spec((1, 64)),
            pl.BlockSpec((1, 1, RB), lambda i: (i, 0, 0)),
            _full_spec((64, 32)), _full_spec((1, 32)),
            _full_spec((32, C)), _full_spec((1, C)),
        ],
        out_specs=_full_spec((G, C)),
        out_shape=jax.ShapeDtypeStruct((G, C), jnp.float32),
        scratch_shapes=[
            pltpu.VMEM((G, 64), jnp.float32),
            pltpu.VMEM((G, 1), jnp.float32),
        ],
    )(p, s, q, g, be, batch2d, fcW1, fcb1, fcW2, fcb2)


def kernel(x, edge_index, batch, W1l, W1r, b1, g1, be1, W2l, W2r, b2, g2, be2,
           W3l, W3r, b3, g3, be3, fcW1, fcb1, fcW2, fcb2):
    src = edge_index[0]
    dst = edge_index[1]

    r1 = lambda v: v.reshape(1, -1)

    aggp1, cntp = _sc_agg(x, src, dst, 128, True)
    p1, cnt, s1, q1 = _tc_a1(aggp1, cntp.reshape(NW, NPAD).T, x,
                             W1l, W1r, r1(b1))
    t2, u2 = _tc_b(p1, s1, q1, r1(g1), r1(be1), W2l, W2r, r1(b2), 256, 128, 128)

    (aggp2,) = _sc_agg(t2, src, dst, 128, False)
    p2, s2, q2 = _tc_a23(aggp2, cnt, u2, 128, 128)
    W3lp = jnp.concatenate([W3l, jnp.zeros((128, 64), jnp.float32)], axis=1)
    t3, u3 = _tc_b(p2, s2, q2, r1(g2), r1(be2), W3lp, W3r, r1(b3), 128, 128, 64)

    (aggp3,) = _sc_agg(t3, src, dst, 128, False)
    p3, s3, q3 = _tc_a23(aggp3, cnt, u3, 128, 64)

    return _tc_c(p3, s3, q3, r1(g3), r1(be3), batch.reshape(GRID, 1, RB),
                 fcW1, r1(fcb1), fcW2, r1(fcb2))
